# baseline (speedup 1.0000x reference)
import functools

import jax
import jax.numpy as jnp
from jax import lax
from jax.experimental import pallas as pl
from jax.experimental.pallas import tpu as pltpu
from jax.experimental.pallas import tpu_sc as plsc

P = 32
ALPHA = 16
VOCAB = 1000000
N_IDS = 1024 * 50
NW = 32
IDS_PER_W = N_IDS // NW
GRP = IDS_PER_W // 16
ELEMS_PER_W = IDS_PER_W * P


def _make_gather():
    mesh = plsc.VectorSubcoreMesh(core_axis_name="c", subcore_axis_name="s")

    @functools.partial(
        pl.kernel,
        mesh=mesh,
        out_type=jax.ShapeDtypeStruct((N_IDS * P,), jnp.float32),
        compiler_params=pltpu.CompilerParams(needs_layout_passes=False),
        scratch_types=[
            pltpu.VMEM((IDS_PER_W,), jnp.int32),
            pltpu.VMEM((ELEMS_PER_W,), jnp.int32),
            pltpu.VMEM((ELEMS_PER_W,), jnp.float32),
            pltpu.SemaphoreType.DMA,
        ],
    )
    def gather(ids_hbm, w_hbm, u_hbm, ids_v, idx_v, u_v, sem):
        wid = lax.axis_index("s") * 2 + lax.axis_index("c")
        base = wid * IDS_PER_W
        pltpu.sync_copy(ids_hbm.at[pl.ds(base, IDS_PER_W)], ids_v)
        lanes = lax.iota(jnp.int32, 16)
        pos0 = lanes * P

        def body(g, carry):
            ids16 = ids_v[pl.ds(pl.multiple_of(g * 16, 16), 16)]
            posg = pos0 + g * (16 * P)
            for p in range(P):
                plsc.store_scatter(idx_v, [posg + p], ids16 + p * VOCAB)
            return carry

        lax.fori_loop(0, GRP, body, 0, unroll=False)
        pltpu.async_copy(w_hbm.at[idx_v], u_v, sem).wait()
        pltpu.sync_copy(u_v, u_hbm.at[pl.ds(wid * ELEMS_PER_W, ELEMS_PER_W)])

    return gather


_gather_call = _make_gather()

_BLK = 512
_NBLK = N_IDS // _BLK


def _dense_body(u_ref, sp_ref, e_ref, fh_ref, y_ref):
    u = u_ref[...]
    m = (u >= sp_ref[...]).astype(jnp.float32)
    fh_ref[...] = m
    y_ref[...] = lax.dot_general(
        m * u, e_ref[...], (((1,), (0,)), ((), ())),
        preferred_element_type=jnp.float32,
        precision=lax.Precision.HIGHEST,
    )


def kernel(input_ids, W_emb, sp, f_param):
    ids_flat = input_ids.reshape(-1).astype(jnp.int32)
    w_flat = W_emb.reshape(-1)
    u_flat = _gather_call(ids_flat, w_flat)
    u2 = u_flat.reshape(N_IDS, P)

    spr = sp.astype(jnp.float32).reshape(1, P)
    eye = jnp.eye(P, dtype=jnp.float32)
    E = (eye[:, :, None] * f_param.astype(jnp.float32)[None, :, :]).reshape(P, P * ALPHA)

    fh2, y2 = pl.pallas_call(
        _dense_body,
        grid=(_NBLK,),
        in_specs=[
            pl.BlockSpec((_BLK, P), lambda i: (i, 0)),
            pl.BlockSpec((1, P), lambda i: (0, 0)),
            pl.BlockSpec((P, P * ALPHA), lambda i: (0, 0)),
        ],
        out_specs=[
            pl.BlockSpec((_BLK, P), lambda i: (i, 0)),
            pl.BlockSpec((_BLK, P * ALPHA), lambda i: (i, 0)),
        ],
        out_shape=[
            jax.ShapeDtypeStruct((N_IDS, P), jnp.float32),
            jax.ShapeDtypeStruct((N_IDS, P * ALPHA), jnp.float32),
        ],
    )(u2, spr, E)

    u = u2.reshape(1024, 50, P)
    f_hard = fh2.reshape(1024, 50, P)
    y = y2.reshape(1024, 50, P, ALPHA)
    return (u, f_hard, y)


# speedup vs baseline: 1.0008x; 1.0008x over previous
import functools

import jax
import jax.numpy as jnp
from jax import lax
from jax.experimental import pallas as pl
from jax.experimental.pallas import tpu as pltpu
from jax.experimental.pallas import tpu_sc as plsc

P = 32
ALPHA = 16
VOCAB = 1000000
N_IDS = 1024 * 50
NW = 32
IDS_PER_W = N_IDS // NW
GRP = IDS_PER_W // 16
ELEMS_PER_W = IDS_PER_W * P
CHUNK = 128
NCHUNK = ELEMS_PER_W // CHUNK


def _make_gather():
    mesh = plsc.VectorSubcoreMesh(core_axis_name="c", subcore_axis_name="s")

    @functools.partial(
        pl.kernel,
        mesh=mesh,
        out_type=jax.ShapeDtypeStruct((N_IDS * P,), jnp.float32),
        compiler_params=pltpu.CompilerParams(needs_layout_passes=False),
        scratch_types=[
            pltpu.VMEM((IDS_PER_W,), jnp.int32),
            pltpu.VMEM((ELEMS_PER_W,), jnp.int32),
            pltpu.VMEM((ELEMS_PER_W,), jnp.float32),
            pltpu.SemaphoreType.DMA,
        ],
    )
    def gather(ids_hbm, w_hbm, u_hbm, ids_v, idx_v, u_v, sem):
        wid = lax.axis_index("s") * 2 + lax.axis_index("c")
        base = wid * IDS_PER_W
        pltpu.sync_copy(ids_hbm.at[pl.ds(base, IDS_PER_W)], ids_v)
        lanes = lax.iota(jnp.int32, 16)
        pos0 = lanes * P

        def body(g, carry):
            ids16 = ids_v[pl.ds(pl.multiple_of(g * 16, 16), 16)]
            posg = pos0 + g * (16 * P)
            for p in range(P):
                plsc.store_scatter(idx_v, [posg + p], ids16 + p * VOCAB)
            return carry

        lax.fori_loop(0, GRP, body, 0, unroll=False)

        def fire(j, carry):
            s = pl.ds(pl.multiple_of(j * CHUNK, CHUNK), CHUNK)
            pltpu.async_copy(w_hbm.at[idx_v.at[s]], u_v.at[s], sem)
            return carry

        def drain(j, carry):
            s = pl.ds(pl.multiple_of(j * CHUNK, CHUNK), CHUNK)
            pltpu.make_async_copy(w_hbm.at[idx_v.at[s]], u_v.at[s], sem).wait()
            return carry

        lax.fori_loop(0, NCHUNK, fire, 0, unroll=False)
        lax.fori_loop(0, NCHUNK, drain, 0, unroll=False)
        pltpu.sync_copy(u_v, u_hbm.at[pl.ds(wid * ELEMS_PER_W, ELEMS_PER_W)])

    return gather


_gather_call = _make_gather()

_BLK = 512
_NBLK = N_IDS // _BLK


def _dense_body(u_ref, sp_ref, e_ref, fh_ref, y_ref):
    u = u_ref[...]
    m = (u >= sp_ref[...]).astype(jnp.float32)
    fh_ref[...] = m
    y_ref[...] = lax.dot_general(
        m * u, e_ref[...], (((1,), (0,)), ((), ())),
        preferred_element_type=jnp.float32,
        precision=lax.Precision.HIGHEST,
    )


def kernel(input_ids, W_emb, sp, f_param):
    ids_flat = input_ids.reshape(-1).astype(jnp.int32)
    w_flat = W_emb.reshape(-1)
    u_flat = _gather_call(ids_flat, w_flat)
    u2 = u_flat.reshape(N_IDS, P)

    spr = sp.astype(jnp.float32).reshape(1, P)
    eye = jnp.eye(P, dtype=jnp.float32)
    E = (eye[:, :, None] * f_param.astype(jnp.float32)[None, :, :]).reshape(P, P * ALPHA)

    fh2, y2 = pl.pallas_call(
        _dense_body,
        grid=(_NBLK,),
        in_specs=[
            pl.BlockSpec((_BLK, P), lambda i: (i, 0)),
            pl.BlockSpec((1, P), lambda i: (0, 0)),
            pl.BlockSpec((P, P * ALPHA), lambda i: (0, 0)),
        ],
        out_specs=[
            pl.BlockSpec((_BLK, P), lambda i: (i, 0)),
            pl.BlockSpec((_BLK, P * ALPHA), lambda i: (i, 0)),
        ],
        out_shape=[
            jax.ShapeDtypeStruct((N_IDS, P), jnp.float32),
            jax.ShapeDtypeStruct((N_IDS, P * ALPHA), jnp.float32),
        ],
    )(u2, spr, E)

    u = u2.reshape(1024, 50, P)
    f_hard = fh2.reshape(1024, 50, P)
    y = y2.reshape(1024, 50, P, ALPHA)
    return (u, f_hard, y)


# speedup vs baseline: 2.5298x; 2.5278x over previous
import functools

import jax
import jax.numpy as jnp
from jax import lax
from jax.experimental import pallas as pl
from jax.experimental.pallas import tpu as pltpu
from jax.experimental.pallas import tpu_sc as plsc

P = 32
ALPHA = 16
VOCAB = 1000000
VOCAB_PAD = 1000064
N_IDS = 1024 * 50
NW = 32
IDS_PER_W = N_IDS // NW
GRP = IDS_PER_W // 16
ELEMS_PER_W = IDS_PER_W * P
CHUNK = 128
NCHUNK = ELEMS_PER_W // CHUNK
W_WORDS = P * VOCAB_PAD


def _make_gather():
    mesh = plsc.VectorSubcoreMesh(core_axis_name="c", subcore_axis_name="s")

    @functools.partial(
        pl.kernel,
        mesh=mesh,
        out_type=jax.ShapeDtypeStruct((N_IDS * P,), jnp.float32),
        compiler_params=pltpu.CompilerParams(needs_layout_passes=False),
        scratch_types=[
            pltpu.VMEM((IDS_PER_W,), jnp.int32),
            pltpu.VMEM((ELEMS_PER_W,), jnp.int32),
            pltpu.VMEM((ELEMS_PER_W,), jnp.float32),
            pltpu.SemaphoreType.DMA,
        ],
    )
    def gather(ids_hbm, w_hbm, u_hbm, ids_v, idx_v, u_v, sem):
        wid = lax.axis_index("s") * 2 + lax.axis_index("c")
        base = wid * IDS_PER_W
        pltpu.sync_copy(ids_hbm.at[pl.ds(base, IDS_PER_W)], ids_v)
        lanes = lax.iota(jnp.int32, 16)
        pos0 = lanes * P

        def body(g, carry):
            ids16 = ids_v[pl.ds(pl.multiple_of(g * 16, 16), 16)]
            posg = pos0 + g * (16 * P)
            for p in range(P):
                plsc.store_scatter(idx_v, [posg + p], ids16 + p * VOCAB_PAD)
            return carry

        lax.fori_loop(0, GRP, body, 0, unroll=False)

        def fire(j, carry):
            s = pl.ds(pl.multiple_of(j * CHUNK, CHUNK), CHUNK)
            pltpu.async_copy(w_hbm.at[idx_v.at[s]], u_v.at[s], sem)
            return carry

        def drain(j, carry):
            s = pl.ds(pl.multiple_of(j * CHUNK, CHUNK), CHUNK)
            pltpu.make_async_copy(w_hbm.at[idx_v.at[s]], u_v.at[s], sem).wait()
            return carry

        lax.fori_loop(0, NCHUNK, fire, 0, unroll=False)
        lax.fori_loop(0, NCHUNK, drain, 0, unroll=False)
        pltpu.sync_copy(u_v, u_hbm.at[pl.ds(wid * ELEMS_PER_W, ELEMS_PER_W)])

    return gather


_gather_call = _make_gather()

_BLK = 256
_NROWS = N_IDS * P // 128
_NBLK = _NROWS // _BLK


def _dense_body(u_ref, sp_ref, e_ref, fh_ref, y_ref):
    u = u_ref[...]
    m = (u >= sp_ref[...]).astype(jnp.float32)
    fh_ref[...] = m
    y_ref[...] = lax.dot_general(
        m * u, e_ref[...], (((1,), (0,)), ((), ())),
        preferred_element_type=jnp.float32,
        precision=lax.Precision.HIGHEST,
    )


def kernel(input_ids, W_emb, sp, f_param):
    ids_flat = input_ids.reshape(-1).astype(jnp.int32)
    w128 = lax.pad(
        W_emb, jnp.float32(0), ((0, 0, 0), (0, VOCAB_PAD - VOCAB, 0))
    ).reshape(W_WORDS // 128, 128)
    w128 = lax.optimization_barrier(w128)
    w_flat = w128.reshape(W_WORDS)
    u_flat = _gather_call(ids_flat, w_flat)
    u128 = u_flat.reshape(_NROWS, 128)

    sp_t = jnp.tile(sp.astype(jnp.float32), 4).reshape(1, 128)
    eye = jnp.eye(P, dtype=jnp.float32)
    E = (eye[:, :, None] * f_param.astype(jnp.float32)[None, :, :]).reshape(P, P * ALPHA)
    E4 = jnp.kron(jnp.eye(4, dtype=jnp.float32), E)

    fh128, y128 = pl.pallas_call(
        _dense_body,
        grid=(_NBLK,),
        in_specs=[
            pl.BlockSpec((_BLK, 128), lambda i: (i, 0)),
            pl.BlockSpec((1, 128), lambda i: (0, 0)),
            pl.BlockSpec((128, 2048), lambda i: (0, 0)),
        ],
        out_specs=[
            pl.BlockSpec((_BLK, 128), lambda i: (i, 0)),
            pl.BlockSpec((_BLK, 2048), lambda i: (i, 0)),
        ],
        out_shape=[
            jax.ShapeDtypeStruct((_NROWS, 128), jnp.float32),
            jax.ShapeDtypeStruct((_NROWS, 2048), jnp.float32),
        ],
    )(u128, sp_t, E4)

    u = u128.reshape(1024, 50, P)
    f_hard = fh128.reshape(1024, 50, P)
    y = y128.reshape(1024, 50, P, ALPHA)
    return (u, f_hard, y)


# speedup vs baseline: 6.9080x; 2.7306x over previous
import functools

import jax
import jax.numpy as jnp
from jax import lax
from jax.experimental import pallas as pl
from jax.experimental.pallas import tpu as pltpu
from jax.experimental.pallas import tpu_sc as plsc

P = 32
ALPHA = 16
VOCAB = 1000000
VOCAB_PAD = 1000064
B = 1024
L = 50
N_IDS = B * L
W_WORDS = P * VOCAB_PAD

NB = 8
NLW = 4
CHUNK = 128
MAXROWS = 13 * P


def _make_gather():
    mesh = plsc.VectorSubcoreMesh(core_axis_name="c", subcore_axis_name="s")

    @functools.partial(
        pl.kernel,
        mesh=mesh,
        out_type=jax.ShapeDtypeStruct((NB, L * P, 128), jnp.float32),
        compiler_params=pltpu.CompilerParams(needs_layout_passes=False),
        scratch_types=[
            pltpu.VMEM((13 * 128,), jnp.int32),
            pltpu.VMEM((MAXROWS * 128,), jnp.int32),
            pltpu.VMEM((MAXROWS, 128), jnp.float32),
            pltpu.SemaphoreType.DMA,
        ],
    )
    def gather(ids_hbm, w_hbm, u_hbm, ids_v, idx_v, u_v, sem):
        wid = lax.axis_index("s") * 2 + lax.axis_index("c")
        wb = wid % NB
        wl = wid // NB

        def phase(l0, nl):
            pltpu.sync_copy(
                ids_hbm.at[pl.ds(wb * (L * 128) + l0 * 128, nl * 128)],
                ids_v.at[pl.ds(0, nl * 128)],
            )

            def build(j, carry):
                l_rel = j // P
                p = j % P
                off = p * VOCAB_PAD
                for h in range(8):
                    ids16 = ids_v[pl.ds(l_rel * 128 + h * 16, 16)]
                    idx_v[pl.ds(j * 128 + h * 16, 16)] = ids16 + off
                return carry

            lax.fori_loop(0, nl * P, build, 0, unroll=False)

            def fire(j, carry):
                s = pl.ds(pl.multiple_of(j * CHUNK, CHUNK), CHUNK)
                pltpu.async_copy(w_hbm.at[idx_v.at[s]], u_v.at[j], sem)
                return carry

            def drain(j, carry):
                s = pl.ds(pl.multiple_of(j * CHUNK, CHUNK), CHUNK)
                pltpu.make_async_copy(
                    w_hbm.at[idx_v.at[s]], u_v.at[j], sem
                ).wait()
                return carry

            lax.fori_loop(0, nl * P, fire, 0, unroll=False)
            lax.fori_loop(0, nl * P, drain, 0, unroll=False)
            pltpu.sync_copy(
                u_v.at[pl.ds(0, nl * P), :],
                u_hbm.at[wb, pl.ds(l0 * P, nl * P), :],
            )

        @pl.when(wl < 2)
        def _():
            phase(wl * 13, 13)

        @pl.when(wl >= 2)
        def _():
            phase(26 + (wl - 2) * 12, 12)

    return gather


_gather_call = _make_gather()

_RB = 64
_NI = L * P // _RB


def _dense_body(u_ref, sp_ref, m_ref, u2_ref, fh_ref, y_ref):
    u = u_ref[0]
    m = (u >= sp_ref[...]).astype(jnp.float32)
    u2_ref[...] = u
    fh_ref[...] = m
    y_ref[...] = lax.dot_general(
        m_ref[...], m * u, (((1,), (0,)), ((), ())),
        preferred_element_type=jnp.float32,
        precision=lax.Precision.HIGHEST,
    )


def kernel(input_ids, W_emb, sp, f_param):
    ids_lin = (
        input_ids.astype(jnp.int32)
        .T.reshape(L, NB, 128)
        .transpose(1, 0, 2)
        .reshape(-1)
    )
    w128 = lax.pad(
        W_emb, jnp.float32(0), ((0, 0, 0), (0, VOCAB_PAD - VOCAB, 0))
    ).reshape(W_WORDS // 128, 128)
    w128 = lax.optimization_barrier(w128)
    w_flat = w128.reshape(W_WORDS)
    u3 = _gather_call(ids_lin, w_flat)

    sp_rep = jnp.tile(sp.astype(jnp.float32), 2).reshape(_RB, 1)
    eye = jnp.eye(P, dtype=jnp.float32)
    E = (eye[:, :, None] * f_param.astype(jnp.float32)[None, :, :]).reshape(
        P, P * ALPHA
    )
    M = jnp.kron(jnp.eye(2, dtype=jnp.float32), E.T)

    u2, fh2, yt = pl.pallas_call(
        _dense_body,
        grid=(_NI, NB),
        in_specs=[
            pl.BlockSpec((1, _RB, 128), lambda i, j: (j, i, 0)),
            pl.BlockSpec((_RB, 1), lambda i, j: (0, 0)),
            pl.BlockSpec((_RB * ALPHA, _RB), lambda i, j: (0, 0)),
        ],
        out_specs=[
            pl.BlockSpec((_RB, 128), lambda i, j: (i, j)),
            pl.BlockSpec((_RB, 128), lambda i, j: (i, j)),
            pl.BlockSpec((_RB * ALPHA, 128), lambda i, j: (i, j)),
        ],
        out_shape=[
            jax.ShapeDtypeStruct((L * P, B), jnp.float32),
            jax.ShapeDtypeStruct((L * P, B), jnp.float32),
            jax.ShapeDtypeStruct((L * P * ALPHA, B), jnp.float32),
        ],
    )(u3, sp_rep, M)

    u = u2.reshape(L, P, B).transpose(2, 0, 1)
    f_hard = fh2.reshape(L, P, B).transpose(2, 0, 1)
    y = yt.reshape(L, P, ALPHA, B).transpose(3, 0, 1, 2)
    return (u, f_hard, y)


# speedup vs baseline: 8.3817x; 1.2133x over previous
import functools

import jax
import jax.numpy as jnp
from jax import lax
from jax.experimental import pallas as pl
from jax.experimental.pallas import tpu as pltpu
from jax.experimental.pallas import tpu_sc as plsc

P = 32
ALPHA = 16
VOCAB = 1000000
VOCAB_PAD = 1000064
B = 1024
L = 50
N_IDS = B * L
W_WORDS = P * VOCAB_PAD

NB = 8
CHUNK = 128
MAXROWS = 13 * P


def _make_gather():
    mesh = plsc.VectorSubcoreMesh(core_axis_name="c", subcore_axis_name="s")

    @functools.partial(
        pl.kernel,
        mesh=mesh,
        out_type=jax.ShapeDtypeStruct((L * P, B), jnp.float32),
        compiler_params=pltpu.CompilerParams(needs_layout_passes=False),
        scratch_types=[
            pltpu.VMEM((13 * 128,), jnp.int32),
            pltpu.VMEM((MAXROWS * 128,), jnp.int32),
            pltpu.VMEM((MAXROWS, 128), jnp.float32),
            pltpu.SemaphoreType.DMA,
        ],
    )
    def gather(ids_hbm, w_hbm, u_hbm, ids_v, idx_v, u_v, sem):
        wid = lax.axis_index("s") * 2 + lax.axis_index("c")
        wb = wid % NB
        wl = wid // NB

        def phase(l0, nl):
            pltpu.sync_copy(
                ids_hbm.at[pl.ds(wb * (L * 128) + l0 * 128, nl * 128)],
                ids_v.at[pl.ds(0, nl * 128)],
            )

            def build_fire(j, carry):
                l_rel = j // P
                p = j % P
                off = p * VOCAB_PAD
                for h in range(8):
                    ids16 = ids_v[pl.ds(l_rel * 128 + h * 16, 16)]
                    idx_v[pl.ds(j * 128 + h * 16, 16)] = ids16 + off
                s = pl.ds(pl.multiple_of(j * CHUNK, CHUNK), CHUNK)
                pltpu.async_copy(w_hbm.at[idx_v.at[s]], u_v.at[j], sem)
                return carry

            def drain(j, carry):
                s = pl.ds(pl.multiple_of(j * CHUNK, CHUNK), CHUNK)
                pltpu.make_async_copy(
                    w_hbm.at[idx_v.at[s]], u_v.at[j], sem
                ).wait()
                return carry

            lax.fori_loop(0, nl * P, build_fire, 0, unroll=False)
            lax.fori_loop(0, nl * P, drain, 0, unroll=False)
            pltpu.sync_copy(
                u_v.at[pl.ds(0, nl * P), :],
                u_hbm.at[pl.ds(l0 * P, nl * P), pl.ds(wb * 128, 128)],
            )

        @pl.when(wl < 2)
        def _():
            phase(wl * 13, 13)

        @pl.when(wl >= 2)
        def _():
            phase(26 + (wl - 2) * 12, 12)

    return gather


_gather_call = _make_gather()

_RB = 64
_NI = L * P // _RB


def _dense_body(u_ref, sp_ref, m_ref, fh_ref, y_ref):
    u = u_ref[...]
    m = (u >= sp_ref[...]).astype(jnp.float32)
    fh_ref[...] = m
    y_ref[...] = lax.dot_general(
        m_ref[...], m * u, (((1,), (0,)), ((), ())),
        preferred_element_type=jnp.float32,
        precision=lax.Precision.HIGHEST,
    )


def kernel(input_ids, W_emb, sp, f_param):
    ids_lin = (
        input_ids.astype(jnp.int32)
        .T.reshape(L, NB, 128)
        .transpose(1, 0, 2)
        .reshape(-1)
    )
    w128 = lax.pad(
        W_emb, jnp.float32(0), ((0, 0, 0), (0, VOCAB_PAD - VOCAB, 0))
    ).reshape(W_WORDS // 128, 128)
    w128 = lax.optimization_barrier(w128)
    w_flat = w128.reshape(W_WORDS)
    u2 = _gather_call(ids_lin, w_flat)

    sp_rep = jnp.tile(sp.astype(jnp.float32), 2).reshape(_RB, 1)
    eye = jnp.eye(P, dtype=jnp.float32)
    E = (eye[:, :, None] * f_param.astype(jnp.float32)[None, :, :]).reshape(
        P, P * ALPHA
    )
    M = jnp.kron(jnp.eye(2, dtype=jnp.float32), E.T)

    fh2, yt = pl.pallas_call(
        _dense_body,
        grid=(_NI,),
        in_specs=[
            pl.BlockSpec((_RB, B), lambda i: (i, 0)),
            pl.BlockSpec((_RB, 1), lambda i: (0, 0)),
            pl.BlockSpec((_RB * ALPHA, _RB), lambda i: (0, 0)),
        ],
        out_specs=[
            pl.BlockSpec((_RB, B), lambda i: (i, 0)),
            pl.BlockSpec((_RB * ALPHA, B), lambda i: (i, 0)),
        ],
        out_shape=[
            jax.ShapeDtypeStruct((L * P, B), jnp.float32),
            jax.ShapeDtypeStruct((L * P * ALPHA, B), jnp.float32),
        ],
    )(u2, sp_rep, M)

    u = u2.reshape(L, P, B).transpose(2, 0, 1)
    f_hard = fh2.reshape(L, P, B).transpose(2, 0, 1)
    y = yt.reshape(L, P, ALPHA, B).transpose(3, 0, 1, 2)
    return (u, f_hard, y)


# speedup vs baseline: 9.5189x; 1.1357x over previous
import functools

import jax
import jax.numpy as jnp
from jax import lax
from jax.experimental import pallas as pl
from jax.experimental.pallas import tpu as pltpu
from jax.experimental.pallas import tpu_sc as plsc

P = 32
ALPHA = 16
VOCAB = 1000000
VOCAB_PAD = 1000064
B = 1024
L = 50
N_IDS = B * L
W_WORDS = P * VOCAB_PAD

NB = 8
CHUNK = 128
MAXROWS = 13 * P


def _make_gather():
    mesh = plsc.VectorSubcoreMesh(core_axis_name="c", subcore_axis_name="s")

    @functools.partial(
        pl.kernel,
        mesh=mesh,
        out_type=jax.ShapeDtypeStruct((L * P, B), jnp.float32),
        compiler_params=pltpu.CompilerParams(needs_layout_passes=False),
        scratch_types=[
            pltpu.VMEM((13 * 128,), jnp.int32),
            pltpu.VMEM((MAXROWS * 128,), jnp.int32),
            pltpu.VMEM((MAXROWS, 128), jnp.float32),
            pltpu.SemaphoreType.DMA,
        ],
    )
    def gather(ids_hbm, w_hbm, u_hbm, ids_v, idx_v, u_v, sem):
        wid = lax.axis_index("s") * 2 + lax.axis_index("c")
        wb = wid % NB
        wl = wid // NB

        def phase(l0, nl):
            pltpu.sync_copy(
                ids_hbm.at[pl.ds(wb * (L * 128) + l0 * 128, nl * 128)],
                ids_v.at[pl.ds(0, nl * 128)],
            )

            def build_fire(j, carry):
                l_rel = j // P
                p = j % P
                off = p * VOCAB_PAD
                for h in range(8):
                    ids16 = ids_v[pl.ds(l_rel * 128 + h * 16, 16)]
                    idx_v[pl.ds(j * 128 + h * 16, 16)] = ids16 + off
                s = pl.ds(pl.multiple_of(j * CHUNK, CHUNK), CHUNK)
                pltpu.async_copy(w_hbm.at[idx_v.at[s]], u_v.at[j], sem)
                return carry

            def drain(j, carry):
                s = pl.ds(pl.multiple_of(j * CHUNK, CHUNK), CHUNK)
                pltpu.make_async_copy(
                    w_hbm.at[idx_v.at[s]], u_v.at[j], sem
                ).wait()
                return carry

            lax.fori_loop(0, nl * P, build_fire, 0, unroll=False)
            lax.fori_loop(0, nl * P, drain, 0, unroll=False)
            pltpu.sync_copy(
                u_v.at[pl.ds(0, nl * P), :],
                u_hbm.at[pl.ds(l0 * P, nl * P), pl.ds(wb * 128, 128)],
            )

        @pl.when(wl < 2)
        def _():
            phase(wl * 13, 13)

        @pl.when(wl >= 2)
        def _():
            phase(26 + (wl - 2) * 12, 12)

    return gather


_gather_call = _make_gather()

_RB = 64
_NI = L * P // _RB


def _dense_body(u_ref, sp_ref, m_ref, fh_ref, y_ref):
    u = u_ref[...]
    m = (u >= sp_ref[...]).astype(jnp.float32)
    fh_ref[...] = m
    y_ref[...] = lax.dot_general(
        m_ref[...], m * u, (((1,), (0,)), ((), ())),
        preferred_element_type=jnp.float32,
    )


def kernel(input_ids, W_emb, sp, f_param):
    ids_lin = (
        input_ids.astype(jnp.int32)
        .T.reshape(L, NB, 128)
        .transpose(1, 0, 2)
        .reshape(-1)
    )
    w128 = lax.pad(
        W_emb, jnp.float32(0), ((0, 0, 0), (0, VOCAB_PAD - VOCAB, 0))
    ).reshape(W_WORDS // 128, 128)
    w128 = lax.optimization_barrier(w128)
    w_flat = w128.reshape(W_WORDS)
    u2 = _gather_call(ids_lin, w_flat)

    sp_rep = jnp.tile(sp.astype(jnp.float32), 2).reshape(_RB, 1)
    eye = jnp.eye(P, dtype=jnp.float32)
    E = (eye[:, :, None] * f_param.astype(jnp.float32)[None, :, :]).reshape(
        P, P * ALPHA
    )
    M = jnp.kron(jnp.eye(2, dtype=jnp.float32), E.T)

    fh2, yt = pl.pallas_call(
        _dense_body,
        grid=(_NI,),
        in_specs=[
            pl.BlockSpec((_RB, B), lambda i: (i, 0)),
            pl.BlockSpec((_RB, 1), lambda i: (0, 0)),
            pl.BlockSpec((_RB * ALPHA, _RB), lambda i: (0, 0)),
        ],
        out_specs=[
            pl.BlockSpec((_RB, B), lambda i: (i, 0)),
            pl.BlockSpec((_RB * ALPHA, B), lambda i: (i, 0)),
        ],
        out_shape=[
            jax.ShapeDtypeStruct((L * P, B), jnp.float32),
            jax.ShapeDtypeStruct((L * P * ALPHA, B), jnp.float32),
        ],
    )(u2, sp_rep, M)

    u = u2.reshape(L, P, B).transpose(2, 0, 1)
    f_hard = fh2.reshape(L, P, B).transpose(2, 0, 1)
    y = yt.reshape(L, P, ALPHA, B).transpose(3, 0, 1, 2)
    return (u, f_hard, y)


# speedup vs baseline: 12.2426x; 1.2861x over previous
import functools

import jax
import jax.numpy as jnp
from jax import lax
from jax.experimental import pallas as pl
from jax.experimental.pallas import tpu as pltpu
from jax.experimental.pallas import tpu_sc as plsc

P = 32
ALPHA = 16
VOCAB = 1000000
VOCAB_PAD = 1000064
B = 1024
L = 50
N_IDS = B * L
W_WORDS = P * VOCAB_PAD

NB = 8
CHUNK = 128
MAXROWS = 13 * P


_DT_CH = 61 * 128
_DT_COLS = 16 * _DT_CH


def _make_detile():
    mesh = plsc.VectorSubcoreMesh(core_axis_name="c", subcore_axis_name="s")

    @functools.partial(
        pl.kernel,
        mesh=mesh,
        out_type=jax.ShapeDtypeStruct((W_WORDS,), jnp.float32),
        compiler_params=pltpu.CompilerParams(needs_layout_passes=False),
        scratch_types=[
            pltpu.VMEM((2, 8, _DT_CH), jnp.float32),
            pltpu.SemaphoreType.DMA,
            pltpu.SemaphoreType.DMA,
        ],
    )
    def detile(w_hbm, wlin_hbm, tmp_v, sem_in, sem_out):
        wid = lax.axis_index("s") * 2 + lax.axis_index("c")
        band = wid % 4
        g = wid // 4
        col0 = g * _DT_COLS

        def in_refs(k, buf, size):
            return (
                w_hbm.at[
                    pl.ds(band * 8, 8), pl.ds(col0 + k * _DT_CH, size)
                ],
                tmp_v.at[buf, :, pl.ds(0, size)],
            )

        def out_refs(k, buf, i, size):
            return (
                tmp_v.at[buf, i, pl.ds(0, size)],
                wlin_hbm.at[
                    pl.ds(
                        (band * 8 + i) * VOCAB_PAD + col0 + k * _DT_CH, size
                    )
                ],
            )

        def run(sizes):
            n = len(sizes)
            pltpu.async_copy(*in_refs(0, 0, sizes[0]), sem_in)
            for k in range(n):
                buf = k % 2
                pltpu.make_async_copy(*in_refs(k, buf, sizes[k]), sem_in).wait()
                for i in range(8):
                    pltpu.async_copy(*out_refs(k, buf, i, sizes[k]), sem_out)
                if k >= 1:
                    for i in range(8):
                        pltpu.make_async_copy(
                            *out_refs(k - 1, 1 - buf, i, sizes[k - 1]), sem_out
                        ).wait()
                if k + 1 < n:
                    pltpu.async_copy(*in_refs(k + 1, 1 - buf, sizes[k + 1]), sem_in)
            for i in range(8):
                pltpu.make_async_copy(
                    *out_refs(n - 1, (n - 1) % 2, i, sizes[n - 1]), sem_out
                ).wait()

        @pl.when(g < 7)
        def _():
            run((_DT_CH,) * 16)

        @pl.when(g == 7)
        def _():
            run((_DT_CH,) * 16 + (512,))

    return detile


_detile_call = _make_detile()


def _make_gather():
    mesh = plsc.VectorSubcoreMesh(core_axis_name="c", subcore_axis_name="s")

    @functools.partial(
        pl.kernel,
        mesh=mesh,
        out_type=jax.ShapeDtypeStruct((L * P, B), jnp.float32),
        compiler_params=pltpu.CompilerParams(needs_layout_passes=False),
        scratch_types=[
            pltpu.VMEM((13 * 128,), jnp.int32),
            pltpu.VMEM((MAXROWS * 128,), jnp.int32),
            pltpu.VMEM((MAXROWS, 128), jnp.float32),
            pltpu.SemaphoreType.DMA,
        ],
    )
    def gather(ids_hbm, w_hbm, u_hbm, ids_v, idx_v, u_v, sem):
        wid = lax.axis_index("s") * 2 + lax.axis_index("c")
        wb = wid % NB
        wl = wid // NB

        def phase(l0, nl):
            pltpu.sync_copy(
                ids_hbm.at[pl.ds(wb * (L * 128) + l0 * 128, nl * 128)],
                ids_v.at[pl.ds(0, nl * 128)],
            )

            def build_fire(j, carry):
                l_rel = j // P
                p = j % P
                off = p * VOCAB_PAD
                for h in range(8):
                    ids16 = ids_v[pl.ds(l_rel * 128 + h * 16, 16)]
                    idx_v[pl.ds(j * 128 + h * 16, 16)] = ids16 + off
                s = pl.ds(pl.multiple_of(j * CHUNK, CHUNK), CHUNK)
                pltpu.async_copy(w_hbm.at[idx_v.at[s]], u_v.at[j], sem)
                return carry

            def drain(j, carry):
                s = pl.ds(pl.multiple_of(j * CHUNK, CHUNK), CHUNK)
                pltpu.make_async_copy(
                    w_hbm.at[idx_v.at[s]], u_v.at[j], sem
                ).wait()
                return carry

            lax.fori_loop(0, nl * P, build_fire, 0, unroll=False)
            lax.fori_loop(0, nl * P, drain, 0, unroll=False)
            pltpu.sync_copy(
                u_v.at[pl.ds(0, nl * P), :],
                u_hbm.at[pl.ds(l0 * P, nl * P), pl.ds(wb * 128, 128)],
            )

        @pl.when(wl < 2)
        def _():
            phase(wl * 13, 13)

        @pl.when(wl >= 2)
        def _():
            phase(26 + (wl - 2) * 12, 12)

    return gather


_gather_call = _make_gather()

_RB = 64
_NI = L * P // _RB


def _dense_body(u_ref, sp_ref, m_ref, fh_ref, y_ref):
    u = u_ref[...]
    m = (u >= sp_ref[...]).astype(jnp.float32)
    fh_ref[...] = m
    y_ref[...] = lax.dot_general(
        m_ref[...], m * u, (((1,), (0,)), ((), ())),
        preferred_element_type=jnp.float32,
    )


def kernel(input_ids, W_emb, sp, f_param):
    ids_lin = (
        input_ids.astype(jnp.int32)
        .T.reshape(L, NB, 128)
        .transpose(1, 0, 2)
        .reshape(-1)
    )
    w_flat = _detile_call(W_emb)
    tail = W_emb[:, 128 * (VOCAB // 128):]
    for p in range(P):
        w_flat = lax.dynamic_update_slice(
            w_flat, tail[p], (p * VOCAB_PAD + 128 * (VOCAB // 128),)
        )
    u2 = _gather_call(ids_lin, w_flat)

    sp_rep = jnp.tile(sp.astype(jnp.float32), 2).reshape(_RB, 1)
    eye = jnp.eye(P, dtype=jnp.float32)
    E = (eye[:, :, None] * f_param.astype(jnp.float32)[None, :, :]).reshape(
        P, P * ALPHA
    )
    M = jnp.kron(jnp.eye(2, dtype=jnp.float32), E.T)

    fh2, yt = pl.pallas_call(
        _dense_body,
        grid=(_NI,),
        in_specs=[
            pl.BlockSpec((_RB, B), lambda i: (i, 0)),
            pl.BlockSpec((_RB, 1), lambda i: (0, 0)),
            pl.BlockSpec((_RB * ALPHA, _RB), lambda i: (0, 0)),
        ],
        out_specs=[
            pl.BlockSpec((_RB, B), lambda i: (i, 0)),
            pl.BlockSpec((_RB * ALPHA, B), lambda i: (i, 0)),
        ],
        out_shape=[
            jax.ShapeDtypeStruct((L * P, B), jnp.float32),
            jax.ShapeDtypeStruct((L * P * ALPHA, B), jnp.float32),
        ],
    )(u2, sp_rep, M)

    u = u2.reshape(L, P, B).transpose(2, 0, 1)
    f_hard = fh2.reshape(L, P, B).transpose(2, 0, 1)
    y = yt.reshape(L, P, ALPHA, B).transpose(3, 0, 1, 2)
    return (u, f_hard, y)
